# baseline (speedup 1.0000x reference)
import functools

import jax
import jax.numpy as jnp
from jax import lax
from jax.experimental import pallas as pl
from jax.experimental.pallas import tpu as pltpu
from jax.experimental.pallas import tpu_sc as plsc

VOCAB = 1000000
EMBED_DIM = 64
BATCH = 1024
SEQ_LEN = 200

_INFO = plsc.get_sparse_core_info()
_NC, _NS, _L = _INFO.num_cores, _INFO.num_subcores, _INFO.num_lanes
_NW = _NC * _NS
_B = BATCH * SEQ_LEN
_BPW = _B // _NW
_CHUNK = SEQ_LEN
_NCHUNK = _BPW // _CHUNK
_SPLIT = 128
_CG = EMBED_DIM // _L


def _sinusoid_table(seq_len, d):
    pos = jnp.arange(seq_len, dtype=jnp.float32)[:, None]
    i = jnp.arange(d, dtype=jnp.float32)[None, :]
    angle = pos / jnp.power(10000.0, 2.0 * jnp.floor(i / 2.0) / d)
    even_mask = (jnp.arange(d) % 2 == 0)[None, :]
    return jnp.where(even_mask, jnp.sin(angle), jnp.cos(angle))


@functools.partial(
    pl.kernel,
    mesh=plsc.VectorSubcoreMesh(core_axis_name="c", subcore_axis_name="s"),
    out_type=jax.ShapeDtypeStruct((_B, EMBED_DIM), jnp.float32),
    scratch_types=[
        pltpu.VMEM((_BPW,), jnp.int32),
        pltpu.VMEM((SEQ_LEN, EMBED_DIM), jnp.float32),
        pltpu.VMEM((_CHUNK, EMBED_DIM), jnp.float32),
        pltpu.SemaphoreType.DMA,
        pltpu.SemaphoreType.DMA,
    ],
    compiler_params=pltpu.CompilerParams(use_tc_tiling_on_sc=False),
)
def _embed_sc(table_hbm, idx_hbm, pe_hbm, out_hbm, idx_v, pe_v, buf, sem_g, sem_s):
    wid = lax.axis_index("s") * _NC + lax.axis_index("c")
    wbase = wid * _BPW
    pltpu.sync_copy(idx_hbm.at[pl.ds(wbase, _BPW)], idx_v)
    pltpu.sync_copy(pe_hbm, pe_v)

    scale = jnp.float32(EMBED_DIM ** -0.5)

    def fma_body(r, carry):
        for cg in range(_CG):
            sl = pl.ds(cg * _L, _L)
            buf[r, sl] = buf[r, sl] * scale + pe_v[r, sl]
        return carry

    for c in range(_NCHUNK):
        base = c * _CHUNK
        d1 = pltpu.async_copy(
            table_hbm.at[idx_v.at[pl.ds(base, _SPLIT)]],
            buf.at[pl.ds(0, _SPLIT)], sem_g)
        d2 = pltpu.async_copy(
            table_hbm.at[idx_v.at[pl.ds(base + _SPLIT, _CHUNK - _SPLIT)]],
            buf.at[pl.ds(_SPLIT, _CHUNK - _SPLIT)], sem_g)
        d1.wait()
        d2.wait()
        lax.fori_loop(0, _CHUNK, fma_body, 0)
        pltpu.async_copy(buf, out_hbm.at[pl.ds(wbase + base, _CHUNK)], sem_s).wait()


def kernel(input, table):
    idx = input.reshape(-1).astype(jnp.int32)
    pe = _sinusoid_table(SEQ_LEN, EMBED_DIM)
    out = _embed_sc(table, idx, pe)
    return out.reshape(BATCH, SEQ_LEN, EMBED_DIM)


# speedup vs baseline: 1.0674x; 1.0674x over previous
import functools

import jax
import jax.numpy as jnp
from jax import lax
from jax.experimental import pallas as pl
from jax.experimental.pallas import tpu as pltpu
from jax.experimental.pallas import tpu_sc as plsc

VOCAB = 1000000
EMBED_DIM = 64
BATCH = 1024
SEQ_LEN = 200

_INFO = plsc.get_sparse_core_info()
_NC, _NS, _L = _INFO.num_cores, _INFO.num_subcores, _INFO.num_lanes
_NW = _NC * _NS
_B = BATCH * SEQ_LEN
_BPW = _B // _NW
_CHUNK = SEQ_LEN
_NCHUNK = _BPW // _CHUNK
_SPLIT = 128
_CG = EMBED_DIM // _L
_NBUF = 4


def _sinusoid_table(seq_len, d):
    pos = jnp.arange(seq_len, dtype=jnp.float32)[:, None]
    i = jnp.arange(d, dtype=jnp.float32)[None, :]
    angle = pos / jnp.power(10000.0, 2.0 * jnp.floor(i / 2.0) / d)
    even_mask = (jnp.arange(d) % 2 == 0)[None, :]
    return jnp.where(even_mask, jnp.sin(angle), jnp.cos(angle))


@functools.partial(
    pl.kernel,
    mesh=plsc.VectorSubcoreMesh(core_axis_name="c", subcore_axis_name="s"),
    out_type=jax.ShapeDtypeStruct((_B, EMBED_DIM), jnp.float32),
    scratch_types=[
        pltpu.VMEM((_BPW,), jnp.int32),
        pltpu.VMEM((SEQ_LEN, EMBED_DIM), jnp.float32),
        [pltpu.VMEM((_CHUNK, EMBED_DIM), jnp.float32) for _ in range(_NBUF)],
        [pltpu.SemaphoreType.DMA for _ in range(_NBUF)],
        [pltpu.SemaphoreType.DMA for _ in range(_NBUF)],
    ],
    compiler_params=pltpu.CompilerParams(use_tc_tiling_on_sc=False),
)
def _embed_sc(table_hbm, idx_hbm, pe_hbm, out_hbm, idx_v, pe_v, bufs, sems_g, sems_s):
    wid = lax.axis_index("s") * _NC + lax.axis_index("c")
    wbase = wid * _BPW
    pltpu.sync_copy(idx_hbm.at[pl.ds(wbase, _BPW)], idx_v)
    pltpu.sync_copy(pe_hbm, pe_v)

    scale = jnp.float32(EMBED_DIM ** -0.5)

    def compute(buf):
        def fma_body(r, carry):
            for cg in range(_CG):
                sl = pl.ds(cg * _L, _L)
                buf[r, sl] = buf[r, sl] * scale + pe_v[r, sl]
            return carry
        lax.fori_loop(0, _CHUNK, fma_body, 0)

    gathers = {}
    stores = {}

    def fire_gather(c):
        b = c % _NBUF
        base = c * _CHUNK
        gathers[c] = (
            pltpu.async_copy(table_hbm.at[idx_v.at[pl.ds(base, _SPLIT)]],
                             bufs[b].at[pl.ds(0, _SPLIT)], sems_g[b]),
            pltpu.async_copy(table_hbm.at[idx_v.at[pl.ds(base + _SPLIT, _CHUNK - _SPLIT)]],
                             bufs[b].at[pl.ds(_SPLIT, _CHUNK - _SPLIT)], sems_g[b]),
        )

    for c in range(_NBUF - 1):
        fire_gather(c)
    for c in range(_NCHUNK):
        b = c % _NBUF
        for d in gathers.pop(c):
            d.wait()
        compute(bufs[b])
        stores[c] = pltpu.async_copy(
            bufs[b], out_hbm.at[pl.ds(wbase + c * _CHUNK, _CHUNK)], sems_s[b])
        n = c + _NBUF - 1
        if n < _NCHUNK:
            if n - _NBUF >= 0:
                stores.pop(n - _NBUF).wait()
            fire_gather(n)
    for c in sorted(stores):
        stores.pop(c).wait()


def kernel(input, table):
    idx = input.reshape(-1).astype(jnp.int32)
    pe = _sinusoid_table(SEQ_LEN, EMBED_DIM)
    out = _embed_sc(table, idx, pe)
    return out.reshape(BATCH, SEQ_LEN, EMBED_DIM)


# speedup vs baseline: 1.4267x; 1.3366x over previous
import functools

import jax
import jax.numpy as jnp
from jax import lax
from jax.experimental import pallas as pl
from jax.experimental.pallas import tpu as pltpu
from jax.experimental.pallas import tpu_sc as plsc

VOCAB = 1000000
EMBED_DIM = 64
BATCH = 1024
SEQ_LEN = 200

_INFO = plsc.get_sparse_core_info()
_NC, _NS, _L = _INFO.num_cores, _INFO.num_subcores, _INFO.num_lanes
_NW = _NC * _NS
_B = BATCH * SEQ_LEN
_BPW = _B // _NW
_CHUNK = 2 * SEQ_LEN
_NCHUNK = _BPW // _CHUNK
_NGRP = _CHUNK // 16
_CG = EMBED_DIM // _L
_NBUF = 2


def _sinusoid_table(seq_len, d):
    pos = jnp.arange(seq_len, dtype=jnp.float32)[:, None]
    i = jnp.arange(d, dtype=jnp.float32)[None, :]
    angle = pos / jnp.power(10000.0, 2.0 * jnp.floor(i / 2.0) / d)
    even_mask = (jnp.arange(d) % 2 == 0)[None, :]
    return jnp.where(even_mask, jnp.sin(angle), jnp.cos(angle))


@functools.partial(
    pl.kernel,
    mesh=plsc.VectorSubcoreMesh(core_axis_name="c", subcore_axis_name="s"),
    out_type=jax.ShapeDtypeStruct((_B, EMBED_DIM), jnp.float32),
    scratch_types=[
        [pltpu.VMEM((_CHUNK,), jnp.int32) for _ in range(_NBUF)],
        pltpu.VMEM((SEQ_LEN * EMBED_DIM,), jnp.float32),
        [pltpu.VMEM((_CHUNK, EMBED_DIM), jnp.float32) for _ in range(_NBUF)],
        [pltpu.SemaphoreType.DMA for _ in range(_NBUF)],
        [pltpu.SemaphoreType.DMA for _ in range(_NBUF)],
        [pltpu.SemaphoreType.DMA for _ in range(_NBUF)],
    ],
    compiler_params=pltpu.CompilerParams(use_tc_tiling_on_sc=True),
)
def _embed_sc(table_hbm, idx_hbm, pe_hbm, out_hbm,
              idx_vms, pe_v, bufs, isems, gsems, ssems):
    wid = lax.axis_index("s") * _NC + lax.axis_index("c")
    wbase = wid * _BPW
    pltpu.sync_copy(pe_hbm, pe_v)

    scale = jnp.float32(EMBED_DIM ** -0.5)

    idx_descs = {}
    store_descs = {}

    def fire_idx(c):
        b = c % _NBUF
        idx_descs[c] = pltpu.async_copy(
            idx_hbm.at[pl.ds(wbase + c * _CHUNK, _CHUNK)], idx_vms[b], isems[b])

    def issue_gathers(c):
        b = c % _NBUF
        idx_sm = idx_vms[b]
        buf = bufs[b]
        sem = gsems[b]

        def body(k, carry):
            base = k * _L
            v = idx_sm[pl.ds(base, _L)]
            for lane in range(_L):
                pltpu.async_copy(table_hbm.at[pl.ds(v[lane], 1), :],
                                 buf.at[pl.ds(base + lane, 1), :], sem)
            return carry
        lax.fori_loop(0, _NGRP, body, 0)

    def drain_gathers(c):
        b = c % _NBUF
        pltpu.make_async_copy(table_hbm.at[pl.ds(0, _CHUNK), :],
                              bufs[b], gsems[b]).wait()

    def compute(c):
        b = c % _NBUF
        buf = bufs[b]

        def fma_body(r, carry):
            for half in range(_CHUNK // SEQ_LEN):
                rr = r + half * SEQ_LEN
                for cg in range(_CG):
                    sl = pl.ds(cg * _L, _L)
                    pe_row = pe_v[pl.ds(r * EMBED_DIM + cg * _L, _L)]
                    buf[rr, sl] = buf[rr, sl] * scale + pe_row
            return carry
        lax.fori_loop(0, SEQ_LEN, fma_body, 0)

    def fire_store(c):
        b = c % _NBUF
        store_descs[c] = pltpu.async_copy(
            bufs[b], out_hbm.at[pl.ds(wbase + c * _CHUNK, _CHUNK), :], ssems[b])

    for j in range(min(_NBUF - 1, _NCHUNK)):
        fire_idx(j)
    for c in range(_NCHUNK + 1):
        if c < _NCHUNK:
            b = c % _NBUF
            if c >= _NBUF:
                store_descs.pop(c - _NBUF).wait()
            idx_descs.pop(c).wait()
            issue_gathers(c)
            if c + _NBUF - 1 < _NCHUNK:
                fire_idx(c + _NBUF - 1)
        if c >= 1:
            drain_gathers(c - 1)
            compute(c - 1)
            fire_store(c - 1)
    for c in sorted(store_descs):
        store_descs.pop(c).wait()


def kernel(input, table):
    idx = input.reshape(-1).astype(jnp.int32)
    pe = _sinusoid_table(SEQ_LEN, EMBED_DIM)
    out = _embed_sc(table, idx, pe.reshape(-1))
    return out.reshape(BATCH, SEQ_LEN, EMBED_DIM)


# speedup vs baseline: 1.8121x; 1.2702x over previous
import functools

import jax
import jax.numpy as jnp
from jax import lax
from jax.experimental import pallas as pl
from jax.experimental.pallas import tpu as pltpu
from jax.experimental.pallas import tpu_sc as plsc

VOCAB = 1000000
EMBED_DIM = 64
BATCH = 1024
SEQ_LEN = 200

_INFO = plsc.get_sparse_core_info()
_NC, _NS, _L = _INFO.num_cores, _INFO.num_subcores, _INFO.num_lanes
_NW = _NC * _NS
_B = BATCH * SEQ_LEN
_BPW = _B // _NW
_CHUNK = 2 * SEQ_LEN
_NCHUNK = _BPW // _CHUNK
_NGRP = _CHUNK // 16
_CG = EMBED_DIM // _L
_NBUF = 2
_G = 8
_CHG = _CHUNK // _G


def _sinusoid_table(seq_len, d):
    pos = jnp.arange(seq_len, dtype=jnp.float32)[:, None]
    i = jnp.arange(d, dtype=jnp.float32)[None, :]
    angle = pos / jnp.power(10000.0, 2.0 * jnp.floor(i / 2.0) / d)
    even_mask = (jnp.arange(d) % 2 == 0)[None, :]
    return jnp.where(even_mask, jnp.sin(angle), jnp.cos(angle))


@functools.partial(
    pl.kernel,
    mesh=plsc.VectorSubcoreMesh(core_axis_name="c", subcore_axis_name="s"),
    out_type=jax.ShapeDtypeStruct((_B // _G, _G, EMBED_DIM), jnp.float32),
    name="embed_gather_sc",
    scratch_types=[
        [pltpu.VMEM((_CHUNK,), jnp.int32) for _ in range(_NBUF)],
        pltpu.VMEM((SEQ_LEN * EMBED_DIM,), jnp.float32),
        [pltpu.VMEM((_CHG, _G, EMBED_DIM), jnp.float32) for _ in range(_NBUF)],
        [pltpu.SemaphoreType.DMA for _ in range(_NBUF)],
        [pltpu.SemaphoreType.DMA for _ in range(_NBUF)],
        [pltpu.SemaphoreType.DMA for _ in range(_NBUF)],
    ],
    compiler_params=pltpu.CompilerParams(use_tc_tiling_on_sc=True),
)
def _embed_sc(table_hbm, idx_hbm, pe_hbm, out_hbm,
              idx_vms, pe_v, bufs, isems, gsems, ssems):
    wid = lax.axis_index("s") * _NC + lax.axis_index("c")
    wbase = wid * _BPW
    pltpu.sync_copy(pe_hbm, pe_v)

    scale = jnp.float32(EMBED_DIM ** -0.5)
    half_g = SEQ_LEN // _G

    def fire_idx(b, c):
        pltpu.async_copy(
            idx_hbm.at[pl.ds(wbase + c * _CHUNK, _CHUNK)], idx_vms[b], isems[b])

    def wait_idx(b):
        pltpu.make_async_copy(idx_hbm.at[pl.ds(0, _CHUNK)],
                              idx_vms[b], isems[b]).wait()

    def issue_gathers(b):
        idx_vm = idx_vms[b]
        buf = bufs[b]
        sem = gsems[b]

        def body(k, carry):
            base = k * _L
            v = idx_vm[pl.ds(base, _L)]
            vg = lax.shift_right_logical(v, 3)
            vs = lax.bitwise_and(v, 7)
            for lane in range(_L):
                dg = 2 * k + lane // _G
                pltpu.async_copy(table_hbm.at[vg[lane], vs[lane], :],
                                 buf.at[dg, lane % _G, :], sem)
            return carry
        lax.fori_loop(0, _NGRP, body, 0)

    def drain_gathers(b):
        pltpu.make_async_copy(table_hbm.at[pl.ds(0, _CHG), :, :],
                              bufs[b], gsems[b]).wait()

    def compute(b):
        buf = bufs[b]

        def fma_body(q, carry):
            g = lax.shift_right_logical(q, 3)
            s = lax.bitwise_and(q, 7)
            for half in range(_CHUNK // SEQ_LEN):
                gg = g + half * half_g
                for cg in range(_CG):
                    sl = pl.ds(cg * _L, _L)
                    pe_row = pe_v[pl.ds(q * EMBED_DIM + cg * _L, _L)]
                    buf[gg, s, sl] = buf[gg, s, sl] * scale + pe_row
            return carry
        lax.fori_loop(0, SEQ_LEN, fma_body, 0)

    def fire_store(b, c):
        pltpu.async_copy(
            bufs[b],
            out_hbm.at[pl.ds((wbase + c * _CHUNK) // _G, _CHG), :, :],
            ssems[b])

    def wait_store(b):
        pltpu.make_async_copy(bufs[b],
                              out_hbm.at[pl.ds(0, _CHG), :, :], ssems[b]).wait()

    fire_idx(0, 0)
    fire_idx(1, 1)
    wait_idx(0)
    issue_gathers(0)
    fire_idx(0, 2)
    wait_idx(1)
    issue_gathers(1)
    fire_idx(1, 3)
    drain_gathers(0)
    compute(0)
    fire_store(0, 0)

    def pair_body(p, carry):
        for b in range(2):
            c = 2 * p + b
            wait_store(b)
            wait_idx(b)
            issue_gathers(b)
            fire_idx(b, c + 2)
            drain_gathers(1 - b)
            compute(1 - b)
            fire_store(1 - b, c - 1)
        return carry
    lax.fori_loop(1, _NCHUNK // 2 - 1, pair_body, 0)

    for c in (_NCHUNK - 2, _NCHUNK - 1):
        b = c % 2
        wait_store(b)
        wait_idx(b)
        issue_gathers(b)
        drain_gathers(1 - b)
        compute(1 - b)
        fire_store(1 - b, c - 1)
    drain_gathers((_NCHUNK - 1) % 2)
    compute((_NCHUNK - 1) % 2)
    fire_store((_NCHUNK - 1) % 2, _NCHUNK - 1)
    wait_store(0)
    wait_store(1)


def kernel(input, table):
    idx = input.reshape(-1).astype(jnp.int32)
    pe = _sinusoid_table(SEQ_LEN, EMBED_DIM)
    table3 = table.reshape(VOCAB // _G, _G, EMBED_DIM)
    out = _embed_sc(table3, idx, pe.reshape(-1))
    return out.reshape(BATCH, SEQ_LEN, EMBED_DIM)
